# Initial kernel scaffold; baseline (speedup 1.0000x reference)
import jax
import jax.numpy as jnp
from jax.experimental import pallas as pl


def kernel(encoder_hs, word_recovery, word_recovery_mask, clause_num_mask, adj_matrix, target_labels, gc1_w, gc1_b, gc2_w, gc2_b, proj_w, proj_b, q_w, q_b, k_w, k_b, v_w, v_b, ao_w, ao_b, ln1_g, ln1_b, int_w, int_b, out_w, out_b, ln2_g, ln2_b, dec_w, dec_b):
    raise NotImplementedError("write your pallas kernel here")


# speedup vs baseline: 2.9008x; 2.9008x over previous
import math

import jax
import jax.numpy as jnp
from jax.experimental import pallas as pl
from jax.experimental.pallas import tpu as pltpu

B, S, H, M, LC, NL, I, NH = 16, 512, 768, 32, 32, 7, 3072, 8
DH = H // NH
LN_EPS = 1e-12
MLC = M * LC

_f32 = jnp.float32
_CDIM0 = (((0,), (0,)), ((), ()))


def _dot(a, b):
  return jnp.dot(a, b, preferred_element_type=_f32)


def _gcn_kernel(enc_ref, wr_ref, wrm_ref, wrmm_ref, adj_ref,
                gc1_w_ref, gc1_b_ref, gc2_w_ref, gc2_b_ref,
                proj_w_ref, proj_b_ref,
                cv_ref,
                ohm_s, xs, s1, s2, ms, pooled_s):
  wr = wr_ref[0]
  wrm = wrm_ref[0]
  iota_s = jax.lax.broadcasted_iota(jnp.int32, (S, MLC), 0)
  ohm_s[...] = jnp.where(iota_s == wr, jnp.broadcast_to(wrm, (S, MLC)), 0.0)

  xs[...] = jax.lax.dot_general(ohm_s[...], enc_ref[0], _CDIM0,
                                preferred_element_type=_f32)
  ones = jnp.ones((S, 128), _f32)
  ms[...] = jax.lax.dot_general(ohm_s[...], ones, _CDIM0,
                                preferred_element_type=_f32)

  s1[...] = _dot(xs[...], gc1_w_ref[...])
  b1 = gc1_b_ref[...]
  for m in range(M):
    sl = slice(m * LC, (m + 1) * LC)
    s2[sl, :] = jnp.maximum(_dot(adj_ref[0, m], s1[sl, :]) + b1, 0.0)

  s1[...] = _dot(s2[...], gc2_w_ref[...])
  b2 = gc2_b_ref[...]
  for m in range(M):
    sl = slice(m * LC, (m + 1) * LC)
    h2m = jnp.maximum(_dot(adj_ref[0, m], s1[sl, :]) + b2, 0.0)
    h2m = h2m * pltpu.repeat(ms[sl, :], H // 128, axis=1)
    xm = xs[sl, :]
    row = jnp.concatenate([
        jnp.max(xm, axis=0, keepdims=True),
        jnp.max(h2m, axis=0, keepdims=True),
        jnp.sum(xm, axis=0, keepdims=True),
        jnp.sum(h2m, axis=0, keepdims=True),
    ], axis=1)
    pooled_s[m:m + 1, :] = row

  lens = jnp.sum(wrmm_ref[0], axis=1, keepdims=True) + 1e-45
  pooled = pooled_s[...]
  pooled = jnp.concatenate([pooled[:, :2 * H], pooled[:, 2 * H:] / lens],
                           axis=1)
  cv_ref[0] = jnp.maximum(_dot(pooled, proj_w_ref[...]) + proj_b_ref[...],
                          0.0)


def _attn_kernel(cv_ref, qw_ref, qb_ref, kw_ref, kb_ref, vw_ref, vb_ref,
                 aow_ref, aob_ref, ln1g_ref, ln1b_ref, cnm_ref,
                 attn_ref, qs, ks, vs, ctx_s):
  cv = cv_ref[...]
  rows = cv.shape[0]
  nb = rows // M
  qs[...] = _dot(cv, qw_ref[...]) + qb_ref[...]
  ks[...] = _dot(cv, kw_ref[...]) + kb_ref[...]
  vs[...] = _dot(cv, vw_ref[...]) + vb_ref[...]
  scale = 1.0 / math.sqrt(DH)
  for bb in range(nb):
    sl = slice(bb * M, (bb + 1) * M)
    amask = (1.0 - cnm_ref[bb]) * (-10000.0)
    qb_ = qs[sl, :]
    kb_ = ks[sl, :]
    vb_ = vs[sl, :]
    parts = []
    for h in range(NH):
      hs = slice(h * DH, (h + 1) * DH)
      sc = jax.lax.dot_general(qb_[:, hs], kb_[:, hs],
                               (((1,), (1,)), ((), ())),
                               preferred_element_type=_f32) * scale + amask
      sc = sc - jnp.max(sc, axis=1, keepdims=True)
      e = jnp.exp(sc)
      att = e / jnp.sum(e, axis=1, keepdims=True)
      parts.append(_dot(att, vb_[:, hs]))
    ctx_s[sl, :] = jnp.concatenate(parts, axis=1)
  co = _dot(ctx_s[...], aow_ref[...]) + aob_ref[...] + cv
  mu = jnp.mean(co, axis=1, keepdims=True)
  d = co - mu
  var = jnp.mean(d * d, axis=1, keepdims=True)
  attn_ref[...] = (d * jax.lax.rsqrt(var + LN_EPS) * ln1g_ref[...]
                   + ln1b_ref[...])


def _ffn_kernel(attn_ref, intw_ref, intb_ref, outw_ref, outb_ref,
                ln2g_ref, ln2b_ref, decw_ref, decb_ref, tgt_ref, cnmc_ref,
                kl_ref, cn_ref, inter_s):
  attn = attn_ref[...]
  rows = attn.shape[0]
  z = _dot(attn, intw_ref[...]) + intb_ref[...]
  inter_s[...] = z * 0.5 * (1.0 + jax.lax.erf(z * (1.0 / math.sqrt(2.0))))
  o = _dot(inter_s[...], outw_ref[...]) + outb_ref[...] + attn
  mu = jnp.mean(o, axis=1, keepdims=True)
  d = o - mu
  var = jnp.mean(d * d, axis=1, keepdims=True)
  out = d * jax.lax.rsqrt(var + LN_EPS) * ln2g_ref[...] + ln2b_ref[...]
  pred = _dot(out, decw_ref[...]) + decb_ref[...]
  mx = jnp.max(pred, axis=1, keepdims=True)
  e = jnp.exp(pred - mx)
  lse = jnp.log(jnp.sum(e, axis=1, keepdims=True)) + mx
  logp = pred - lse
  t = tgt_ref[...].reshape(rows, NL)
  kl_el = jnp.where(t > 0, t * jnp.log(jnp.where(t > 0, t, 1.0)), 0.0) \
      - t * logp
  klc = jnp.sum(kl_el, axis=1, keepdims=True) * (1.0 / NL)
  cnm = cnmc_ref[...].reshape(rows, 1)
  kl_sum = jnp.sum(klc * cnm, axis=0, keepdims=True)
  cn_sum = jnp.sum(cnm, axis=0, keepdims=True)
  kl_ref[...] = jnp.broadcast_to(kl_sum.reshape(1, 1, 1), (1, 1, 128))
  cn_ref[...] = jnp.broadcast_to(cn_sum.reshape(1, 1, 1), (1, 1, 128))


def _loss_kernel(kl_ref, cn_ref, out_ref):
  kl = jnp.sum(kl_ref[:, 0, 0:1], axis=0, keepdims=True)
  cn = jnp.sum(cn_ref[:, 0, 0:1], axis=0, keepdims=True)
  out_ref[...] = kl / cn


@jax.jit
def kernel(encoder_hs, word_recovery, word_recovery_mask, clause_num_mask,
           adj_matrix, target_labels,
           gc1_w, gc1_b, gc2_w, gc2_b, proj_w, proj_b,
           q_w, q_b, k_w, k_b, v_w, v_b, ao_w, ao_b, ln1_g, ln1_b,
           int_w, int_b, out_w, out_b, ln2_g, ln2_b, dec_w, dec_b):
  wr_flat = word_recovery.reshape(B, 1, MLC)
  wrm_row = word_recovery_mask.astype(_f32).reshape(B, 1, MLC)
  wrm_mat = word_recovery_mask.astype(_f32)
  cnm_row = clause_num_mask.astype(_f32).reshape(B, 1, M)
  cnm_col = clause_num_mask.astype(_f32).reshape(B, M, 1)

  row2 = lambda x: x.reshape(1, -1)
  const2 = lambda b: (0, 0)

  cv = pl.pallas_call(
      _gcn_kernel,
      grid=(B,),
      in_specs=[
          pl.BlockSpec((1, S, H), lambda b: (b, 0, 0)),
          pl.BlockSpec((1, 1, MLC), lambda b: (b, 0, 0)),
          pl.BlockSpec((1, 1, MLC), lambda b: (b, 0, 0)),
          pl.BlockSpec((1, M, LC), lambda b: (b, 0, 0)),
          pl.BlockSpec((1, M, LC, LC), lambda b: (b, 0, 0, 0)),
          pl.BlockSpec((H, H), const2),
          pl.BlockSpec((1, H), const2),
          pl.BlockSpec((H, H), const2),
          pl.BlockSpec((1, H), const2),
          pl.BlockSpec((4 * H, H), const2),
          pl.BlockSpec((1, H), const2),
      ],
      out_specs=pl.BlockSpec((1, M, H), lambda b: (b, 0, 0)),
      out_shape=jax.ShapeDtypeStruct((B, M, H), _f32),
      scratch_shapes=[
          pltpu.VMEM((S, MLC), _f32),
          pltpu.VMEM((MLC, H), _f32),
          pltpu.VMEM((MLC, H), _f32),
          pltpu.VMEM((MLC, H), _f32),
          pltpu.VMEM((MLC, 128), _f32),
          pltpu.VMEM((M, 4 * H), _f32),
      ],
      compiler_params=pltpu.CompilerParams(
          dimension_semantics=("parallel",),
          vmem_limit_bytes=56 * 1024 * 1024,
      ),
  )(encoder_hs, wr_flat, wrm_row, wrm_mat, adj_matrix,
    gc1_w, row2(gc1_b), gc2_w, row2(gc2_b), proj_w, row2(proj_b))

  cv2 = cv.reshape(B * M, H)
  RB = B // 2
  R = RB * M

  attn = pl.pallas_call(
      _attn_kernel,
      grid=(2,),
      in_specs=[
          pl.BlockSpec((R, H), lambda c: (c, 0)),
          pl.BlockSpec((H, H), const2),
          pl.BlockSpec((1, H), const2),
          pl.BlockSpec((H, H), const2),
          pl.BlockSpec((1, H), const2),
          pl.BlockSpec((H, H), const2),
          pl.BlockSpec((1, H), const2),
          pl.BlockSpec((H, H), const2),
          pl.BlockSpec((1, H), const2),
          pl.BlockSpec((1, H), const2),
          pl.BlockSpec((1, H), const2),
          pl.BlockSpec((RB, 1, M), lambda c: (c, 0, 0)),
      ],
      out_specs=pl.BlockSpec((R, H), lambda c: (c, 0)),
      out_shape=jax.ShapeDtypeStruct((B * M, H), _f32),
      scratch_shapes=[
          pltpu.VMEM((R, H), _f32),
          pltpu.VMEM((R, H), _f32),
          pltpu.VMEM((R, H), _f32),
          pltpu.VMEM((R, H), _f32),
      ],
      compiler_params=pltpu.CompilerParams(
          dimension_semantics=("parallel",),
          vmem_limit_bytes=56 * 1024 * 1024,
      ),
  )(cv2, q_w, row2(q_b), k_w, row2(k_b), v_w, row2(v_b),
    ao_w, row2(ao_b), row2(ln1_g), row2(ln1_b), cnm_row)

  kl_parts, cn_parts = pl.pallas_call(
      _ffn_kernel,
      grid=(2,),
      in_specs=[
          pl.BlockSpec((R, H), lambda c: (c, 0)),
          pl.BlockSpec((H, I), const2),
          pl.BlockSpec((1, I), const2),
          pl.BlockSpec((I, H), const2),
          pl.BlockSpec((1, H), const2),
          pl.BlockSpec((1, H), const2),
          pl.BlockSpec((1, H), const2),
          pl.BlockSpec((H, NL), const2),
          pl.BlockSpec((1, NL), const2),
          pl.BlockSpec((RB, M, NL), lambda c: (c, 0, 0)),
          pl.BlockSpec((RB, M, 1), lambda c: (c, 0, 0)),
      ],
      out_specs=[
          pl.BlockSpec((1, 1, 128), lambda c: (c, 0, 0)),
          pl.BlockSpec((1, 1, 128), lambda c: (c, 0, 0)),
      ],
      out_shape=[
          jax.ShapeDtypeStruct((2, 1, 128), _f32),
          jax.ShapeDtypeStruct((2, 1, 128), _f32),
      ],
      scratch_shapes=[
          pltpu.VMEM((R, I), _f32),
      ],
      compiler_params=pltpu.CompilerParams(
          dimension_semantics=("parallel",),
          vmem_limit_bytes=56 * 1024 * 1024,
      ),
  )(attn, int_w, row2(int_b), out_w, row2(out_b), row2(ln2_g), row2(ln2_b),
    dec_w, row2(dec_b), target_labels, cnm_col)

  loss = pl.pallas_call(
      _loss_kernel,
      out_shape=jax.ShapeDtypeStruct((1, 1), _f32),
  )(kl_parts, cn_parts)
  return loss.reshape(())
